# baseline (device time: 104162 ns/iter reference)
import jax
import jax.numpy as jnp
from jax import lax
from jax.experimental import pallas as pl
from jax.experimental.pallas import tpu as pltpu

N_DEV = 8


def kernel(x, router_W, route_idx, expert_W, shared_W):
    n_tok, d = x.shape
    e_per, _, h_dim = expert_W.shape
    n_exp = N_DEV * e_per

    def body(x_ref, rw_ref, idx_ref, ew_ref, sw_ref, out_ref,
             comm_ref, send_sems, recv_sems):
        my = lax.axis_index("i")
        left = lax.rem(my - 1 + N_DEV, N_DEV)
        right = lax.rem(my + 1, N_DEV)

        barrier_sem = pltpu.get_barrier_semaphore()
        for nbr in (left, right):
            pl.semaphore_signal(
                barrier_sem, inc=1,
                device_id=(nbr,), device_id_type=pl.DeviceIdType.MESH,
            )
        pl.semaphore_wait(barrier_sem, 2)

        xf = x_ref[...]
        xbf = xf.astype(jnp.bfloat16)

        scores = jnp.dot(xf, rw_ref[...], preferred_element_type=jnp.float32)
        m = jnp.max(scores, axis=-1, keepdims=True)
        p = jnp.exp(scores - m)
        p = p / jnp.sum(p, axis=-1, keepdims=True)

        idx = idx_ref[...]
        col = lax.broadcasted_iota(jnp.int32, (n_tok, n_exp), 1)
        p_top = jnp.sum(jnp.where(col == idx, p, 0.0), axis=-1, keepdims=True)

        acc = jnp.dot(xbf, sw_ref[...].astype(jnp.bfloat16),
                      preferred_element_type=jnp.float32)

        comm_ref[0, :, :, :] = ew_ref[...].astype(jnp.bfloat16)

        for hop in range(N_DEV):
            if hop < N_DEV - 1:
                rdma = pltpu.make_async_remote_copy(
                    src_ref=comm_ref.at[hop],
                    dst_ref=comm_ref.at[hop + 1],
                    send_sem=send_sems.at[hop],
                    recv_sem=recv_sems.at[hop],
                    device_id=(right,),
                    device_id_type=pl.DeviceIdType.MESH,
                )
                rdma.start()
            src_dev = lax.rem(my - hop + N_DEV, N_DEV)
            for j in range(e_per):
                e = src_dev * e_per + j
                coef = jnp.where(idx == e, p_top, 0.0)
                xe = (coef * xf).astype(jnp.bfloat16)
                acc = acc + jnp.dot(xe, comm_ref[hop, j],
                                    preferred_element_type=jnp.float32)
            if hop < N_DEV - 1:
                rdma.wait()

        out_ref[...] = acc

    return pl.pallas_call(
        body,
        out_shape=jax.ShapeDtypeStruct((n_tok, h_dim), jnp.float32),
        in_specs=[pl.BlockSpec(memory_space=pltpu.VMEM)] * 5,
        out_specs=pl.BlockSpec(memory_space=pltpu.VMEM),
        scratch_shapes=[
            pltpu.VMEM((N_DEV, e_per, d, h_dim), jnp.bfloat16),
            pltpu.SemaphoreType.DMA((N_DEV - 1,)),
            pltpu.SemaphoreType.DMA((N_DEV - 1,)),
        ],
        compiler_params=pltpu.CompilerParams(collective_id=0),
    )(x, router_W, route_idx, expert_W, shared_W)


# device time: 65009 ns/iter; 1.6023x vs baseline; 1.6023x over previous
import jax
import jax.numpy as jnp
from jax import lax
from jax.experimental import pallas as pl
from jax.experimental.pallas import tpu as pltpu

N_DEV = 8


def kernel(x, router_W, route_idx, expert_W, shared_W):
    n_tok, d = x.shape
    e_per, _, h_dim = expert_W.shape
    n_exp = N_DEV * e_per

    def body(x_ref, rw_ref, idx_ref, ew_ref, sw_ref, out_ref,
             comm_ref, send_sems, recv_sems):
        my = lax.axis_index("i")
        left = lax.rem(my - 1 + N_DEV, N_DEV)
        right = lax.rem(my + 1, N_DEV)

        barrier_sem = pltpu.get_barrier_semaphore()
        for nbr in (left, right):
            pl.semaphore_signal(
                barrier_sem, inc=1,
                device_id=(nbr,), device_id_type=pl.DeviceIdType.MESH,
            )
        pl.semaphore_wait(barrier_sem, 2)

        xf = x_ref[...]
        xbf = xf.astype(jnp.bfloat16)

        scores = jnp.dot(xf, rw_ref[...], preferred_element_type=jnp.float32)
        m = jnp.max(scores, axis=-1, keepdims=True)
        p = jnp.exp(scores - m)
        p = p / jnp.sum(p, axis=-1, keepdims=True)

        idx = idx_ref[...]
        col = lax.broadcasted_iota(jnp.int32, (n_tok, n_exp), 1)
        p_top = jnp.sum(jnp.where(col == idx, p, 0.0), axis=-1, keepdims=True)

        acc = jnp.dot(xbf, sw_ref[...].astype(jnp.bfloat16),
                      preferred_element_type=jnp.float32)

        comm_ref[0, :, :, :] = ew_ref[...].astype(jnp.bfloat16)

        def compute_slot(slot, src_dev, acc):
            for j in range(e_per):
                e = src_dev * e_per + j
                coef = jnp.where(idx == e, p_top, 0.0)
                xe = (coef * xf).astype(jnp.bfloat16)
                acc = acc + jnp.dot(xe, comm_ref[slot, j],
                                    preferred_element_type=jnp.float32)
            return acc

        N_CW = 4
        N_CCW = 3

        cw = [
            pltpu.make_async_remote_copy(
                src_ref=comm_ref.at[h - 1],
                dst_ref=comm_ref.at[h],
                send_sem=send_sems.at[h - 1],
                recv_sem=recv_sems.at[h - 1],
                device_id=(right,),
                device_id_type=pl.DeviceIdType.MESH,
            )
            for h in range(1, N_CW + 1)
        ]
        ccw = [
            pltpu.make_async_remote_copy(
                src_ref=comm_ref.at[0 if h == 1 else 4 + h - 1],
                dst_ref=comm_ref.at[4 + h],
                send_sem=send_sems.at[N_CW + h - 1],
                recv_sem=recv_sems.at[N_CW + h - 1],
                device_id=(left,),
                device_id_type=pl.DeviceIdType.MESH,
            )
            for h in range(1, N_CCW + 1)
        ]

        cw[0].start()
        ccw[0].start()
        acc = compute_slot(0, my, acc)

        for step in range(1, N_CW + 1):
            cw[step - 1].wait_recv()
            if step < N_CW:
                cw[step].start()
            if step <= N_CCW:
                ccw[step - 1].wait_recv()
                if step < N_CCW:
                    ccw[step].start()
            acc = compute_slot(step, lax.rem(my - step + N_DEV, N_DEV), acc)
            if step <= N_CCW:
                acc = compute_slot(4 + step, lax.rem(my + step, N_DEV), acc)

        out_ref[...] = acc

        for r in cw + ccw:
            r.wait_send()

    return pl.pallas_call(
        body,
        out_shape=jax.ShapeDtypeStruct((n_tok, h_dim), jnp.float32),
        in_specs=[pl.BlockSpec(memory_space=pltpu.VMEM)] * 5,
        out_specs=pl.BlockSpec(memory_space=pltpu.VMEM),
        scratch_shapes=[
            pltpu.VMEM((N_DEV, e_per, d, h_dim), jnp.bfloat16),
            pltpu.SemaphoreType.DMA((N_DEV - 1,)),
            pltpu.SemaphoreType.DMA((N_DEV - 1,)),
        ],
        compiler_params=pltpu.CompilerParams(collective_id=0),
    )(x, router_W, route_idx, expert_W, shared_W)


# device time: 64947 ns/iter; 1.6038x vs baseline; 1.0010x over previous
import jax
import jax.numpy as jnp
from jax import lax
from jax.experimental import pallas as pl
from jax.experimental.pallas import tpu as pltpu

N_DEV = 8


def kernel(x, router_W, route_idx, expert_W, shared_W):
    n_tok, d = x.shape
    e_per, _, h_dim = expert_W.shape
    n_exp = N_DEV * e_per

    def body(x_ref, rw_ref, idx_ref, ew_ref, sw_ref, out_ref,
             comm_ref, send_sems, recv_sems):
        my = lax.axis_index("i")
        left = lax.rem(my - 1 + N_DEV, N_DEV)
        right = lax.rem(my + 1, N_DEV)

        barrier_sem = pltpu.get_barrier_semaphore()
        for nbr in (left, right):
            pl.semaphore_signal(
                barrier_sem, inc=1,
                device_id=(nbr,), device_id_type=pl.DeviceIdType.MESH,
            )
        pl.semaphore_wait(barrier_sem, 2)

        xf = x_ref[...]
        xbf = xf.astype(jnp.bfloat16)

        scores = jnp.dot(xf, rw_ref[...], preferred_element_type=jnp.float32)
        m = jnp.max(scores, axis=-1, keepdims=True)
        p = jnp.exp(scores - m)
        p = p / jnp.sum(p, axis=-1, keepdims=True)

        idx = idx_ref[...]
        col = lax.broadcasted_iota(jnp.int32, (n_tok, n_exp), 1)
        p_top = jnp.sum(jnp.where(col == idx, p, 0.0), axis=-1, keepdims=True)

        acc = jnp.dot(xbf, sw_ref[...].astype(jnp.bfloat16),
                      preferred_element_type=jnp.float32)

        xs_bf = (p_top * xf).astype(jnp.bfloat16)
        zero_bf = jnp.zeros_like(xs_bf)

        comm_ref[0, :, :, :] = ew_ref[...].astype(jnp.bfloat16)

        def compute_slot(slot, src_dev, acc):
            for j in range(e_per):
                e = src_dev * e_per + j
                xe = jnp.where(idx == e, xs_bf, zero_bf)
                acc = acc + jnp.dot(xe, comm_ref[slot, j],
                                    preferred_element_type=jnp.float32)
            return acc

        N_CW = 4
        N_CCW = 3

        cw = [
            pltpu.make_async_remote_copy(
                src_ref=comm_ref.at[h - 1],
                dst_ref=comm_ref.at[h],
                send_sem=send_sems.at[h - 1],
                recv_sem=recv_sems.at[h - 1],
                device_id=(right,),
                device_id_type=pl.DeviceIdType.MESH,
            )
            for h in range(1, N_CW + 1)
        ]
        ccw = [
            pltpu.make_async_remote_copy(
                src_ref=comm_ref.at[0 if h == 1 else 4 + h - 1],
                dst_ref=comm_ref.at[4 + h],
                send_sem=send_sems.at[N_CW + h - 1],
                recv_sem=recv_sems.at[N_CW + h - 1],
                device_id=(left,),
                device_id_type=pl.DeviceIdType.MESH,
            )
            for h in range(1, N_CCW + 1)
        ]

        cw[0].start()
        ccw[0].start()
        acc = compute_slot(0, my, acc)

        for step in range(1, N_CW + 1):
            cw[step - 1].wait_recv()
            if step < N_CW:
                cw[step].start()
            if step <= N_CCW:
                ccw[step - 1].wait_recv()
                if step < N_CCW:
                    ccw[step].start()
            acc = compute_slot(step, lax.rem(my - step + N_DEV, N_DEV), acc)
            if step <= N_CCW:
                acc = compute_slot(4 + step, lax.rem(my + step, N_DEV), acc)

        out_ref[...] = acc

        for r in cw + ccw:
            r.wait_send()

    return pl.pallas_call(
        body,
        out_shape=jax.ShapeDtypeStruct((n_tok, h_dim), jnp.float32),
        in_specs=[pl.BlockSpec(memory_space=pltpu.VMEM)] * 5,
        out_specs=pl.BlockSpec(memory_space=pltpu.VMEM),
        scratch_shapes=[
            pltpu.VMEM((N_DEV, e_per, d, h_dim), jnp.bfloat16),
            pltpu.SemaphoreType.DMA((N_DEV - 1,)),
            pltpu.SemaphoreType.DMA((N_DEV - 1,)),
        ],
        compiler_params=pltpu.CompilerParams(collective_id=0),
    )(x, router_W, route_idx, expert_W, shared_W)


# device time: 58841 ns/iter; 1.7702x vs baseline; 1.1038x over previous
import jax
import jax.numpy as jnp
from jax import lax
from jax.experimental import pallas as pl
from jax.experimental.pallas import tpu as pltpu

N_DEV = 8


def kernel(x, router_W, route_idx, expert_W, shared_W):
    n_tok, d = x.shape
    e_per, _, h_dim = expert_W.shape
    n_exp = N_DEV * e_per

    def body(x_ref, rw_ref, idx_ref, ew_ref, sw_ref, out_ref,
             comm_ref, send_sems, recv_sems):
        my = lax.axis_index("i")
        left = lax.rem(my - 1 + N_DEV, N_DEV)
        right = lax.rem(my + 1, N_DEV)

        barrier_sem = pltpu.get_barrier_semaphore()
        for nbr in (left, right):
            pl.semaphore_signal(
                barrier_sem, inc=1,
                device_id=(nbr,), device_id_type=pl.DeviceIdType.MESH,
            )
        pl.semaphore_wait(barrier_sem, 2)

        comm_ref[0, :, :, :] = ew_ref[...].astype(jnp.bfloat16)

        half = e_per // 2
        cw = [
            pltpu.make_async_remote_copy(
                src_ref=(comm_ref.at[h - 1] if h < 4
                         else comm_ref.at[3, pl.ds(0, half)]),
                dst_ref=(comm_ref.at[h] if h < 4
                         else comm_ref.at[4, pl.ds(0, half)]),
                send_sem=send_sems.at[h - 1],
                recv_sem=recv_sems.at[h - 1],
                device_id=(right,),
                device_id_type=pl.DeviceIdType.MESH,
            )
            for h in range(1, 5)
        ]
        ccw = [
            pltpu.make_async_remote_copy(
                src_ref=(comm_ref.at[0] if h == 1
                         else comm_ref.at[4 + h - 1] if h < 4
                         else comm_ref.at[7, pl.ds(half, half)]),
                dst_ref=(comm_ref.at[4 + h] if h < 4
                         else comm_ref.at[4, pl.ds(half, half)]),
                send_sem=send_sems.at[4 + h - 1],
                recv_sem=recv_sems.at[4 + h - 1],
                device_id=(left,),
                device_id_type=pl.DeviceIdType.MESH,
            )
            for h in range(1, 5)
        ]

        cw[0].start()
        ccw[0].start()

        xf = x_ref[...]
        xbf = xf.astype(jnp.bfloat16)

        scores = jnp.dot(xf, rw_ref[...], preferred_element_type=jnp.float32)
        m = jnp.max(scores, axis=-1, keepdims=True)
        p = jnp.exp(scores - m)
        p = p / jnp.sum(p, axis=-1, keepdims=True)

        idx = idx_ref[...]
        col = lax.broadcasted_iota(jnp.int32, (n_tok, n_exp), 1)
        p_top = jnp.sum(jnp.where(col == idx, p, 0.0), axis=-1, keepdims=True)

        acc = jnp.dot(xbf, sw_ref[...].astype(jnp.bfloat16),
                      preferred_element_type=jnp.float32)

        xs_bf = (p_top * xf).astype(jnp.bfloat16)
        zero_bf = jnp.zeros_like(xs_bf)

        def compute_slot(slot, src_dev, acc):
            for j in range(e_per):
                e = src_dev * e_per + j
                xe = jnp.where(idx == e, xs_bf, zero_bf)
                acc = acc + jnp.dot(xe, comm_ref[slot, j],
                                    preferred_element_type=jnp.float32)
            return acc

        acc = compute_slot(0, my, acc)

        for step in range(1, 4):
            cw[step - 1].wait_recv()
            cw[step].start()
            ccw[step - 1].wait_recv()
            ccw[step].start()
            acc = compute_slot(step, lax.rem(my - step + N_DEV, N_DEV), acc)
            acc = compute_slot(4 + step, lax.rem(my + step, N_DEV), acc)

        cw[3].wait_recv()
        ccw[3].wait_recv()
        acc = compute_slot(4, lax.rem(my + 4, N_DEV), acc)

        out_ref[...] = acc

        for r in cw + ccw:
            r.wait_send()

    return pl.pallas_call(
        body,
        out_shape=jax.ShapeDtypeStruct((n_tok, h_dim), jnp.float32),
        in_specs=[pl.BlockSpec(memory_space=pltpu.VMEM)] * 5,
        out_specs=pl.BlockSpec(memory_space=pltpu.VMEM),
        scratch_shapes=[
            pltpu.VMEM((N_DEV, e_per, d, h_dim), jnp.bfloat16),
            pltpu.SemaphoreType.DMA((8,)),
            pltpu.SemaphoreType.DMA((8,)),
        ],
        compiler_params=pltpu.CompilerParams(collective_id=0),
    )(x, router_W, route_idx, expert_W, shared_W)
